# baseline (speedup 1.0000x reference)
import functools

import jax
import jax.numpy as jnp
from jax import lax
from jax.experimental import pallas as pl
from jax.experimental.pallas import tpu as pltpu
from jax.experimental.pallas import tpu_sc as plsc

B, T, F, VOCAB, D = 1024, 50, 26, 1000000, 16
ROW = 1 + D
NBT = B // 128
NBB = B // 64
TPAD = 56
NCHUNK = T * NBB
NW = 32
NSLOT = NCHUNK // NW
IDS = F * 64
NBATCH = IDS // 128
EVR = IDS // 16


@functools.cache
def _build_embed_sc():
  mesh = plsc.VectorSubcoreMesh(core_axis_name="c", subcore_axis_name="s")
  return functools.partial(
      pl.kernel,
      out_type=jax.ShapeDtypeStruct((ROW, F, TPAD, B), jnp.float32),
      mesh=mesh,
      compiler_params=pltpu.CompilerParams(
          needs_layout_passes=False, use_tc_tiling_on_sc=False),
      scratch_types=[
          pltpu.VMEM((F, 2, 64), jnp.float32),
          pltpu.VMEM((IDS, 16), jnp.float32),
          pltpu.VMEM((IDS, 17), jnp.float32),
          pltpu.VMEM((ROW, F, 64), jnp.float32),
          pltpu.VMEM((NBATCH, 128), jnp.int32),
          pltpu.SemaphoreType.DMA,
          pltpu.SemaphoreType.DMA,
      ],
  )(_embed_sc)


def _embed_sc(x_hbm, tab_hbm, out_hbm, xbuf, rows, rows17, obuf, idxb,
              sem, semo):
  w = lax.axis_index("s") * 2 + lax.axis_index("c")
  rowv0 = lax.iota(jnp.int32, 16)

  def prep(cid):
    @pl.when(cid < NCHUNK)
    def _():
      t = cid // NBB
      bt = (cid % NBB) // 2
      h = cid % 2
      pltpu.sync_copy(x_hbm.at[t, :, bt, :, h, :], xbuf)
      for k in range(EVR):
        f, j = k // 4, k % 4
        v = xbuf[f, 1, pl.ds(j * 16, 16)]
        idxb[k // 8, pl.ds((k % 8) * 16, 16)] = v.astype(jnp.int32)
      for g in range(NBATCH):
        pltpu.async_copy(
            tab_hbm.at[idxb.at[g]], rows.at[pl.ds(g * 128, 128)], sem)

  def body(s, carry):
    cid = s * NW + w
    t = cid // NBB
    bb = cid % NBB

    for g in range(NBATCH):
      pltpu.make_async_copy(
          tab_hbm.at[idxb.at[g]], rows.at[pl.ds(g * 128, 128)], sem).wait()

    def rbody(q, rc):
      for u in range(8):
        sl = q * 8 + u
        rows17[sl, pl.ds(0, 16)] = rows[sl, :]
      return rc

    lax.fori_loop(0, IDS // 8, rbody, 0)

    for k in range(EVR):
      f, j = k // 4, k % 4
      obuf[0, f, pl.ds(j * 16, 16)] = xbuf[f, 0, pl.ds(j * 16, 16)]

    def mbody(mm, mc):
      for u in range(2):
        m = mm * 2 + u
        rowv = rowv0 + m * 16
        f = m // 4
        col = (m % 4) * 16
        for r in range(1, ROW):
          v = plsc.load_gather(
              rows17, [rowv, jnp.full((16,), r - 1, jnp.int32)])
          obuf[r, f, pl.ds(col, 16)] = v
      return mc

    lax.fori_loop(0, EVR // 2, mbody, 0)
    pltpu.async_copy(obuf, out_hbm.at[:, :, t, pl.ds(bb * 64, 64)], semo)

    prep(cid + NW)

    pltpu.make_async_copy(
        obuf, out_hbm.at[:, :, t, pl.ds(bb * 64, 64)], semo).wait()
    return carry

  prep(w)
  lax.fori_loop(0, NSLOT, body, 0)


def kernel(X, table):
  x6 = (X.transpose(1, 3, 2, 0)
          .reshape(T, F, 2, NBT, 128)
          .transpose(0, 1, 3, 2, 4)
          .reshape(T, F, NBT, 2, 2, 64))
  out = _build_embed_sc()(x6, table)
  return out.transpose(3, 2, 0, 1)[:, :T]
